# baseline (speedup 1.0000x reference)
import functools

import jax
import jax.numpy as jnp
from jax import lax
from jax.experimental import pallas as pl
from jax.experimental.pallas import tpu as pltpu
from jax.experimental.pallas import tpu_sc as plsc

N = 10000
E = 320000
E2 = E // 2
D = 128
DH = D // 2
DR = 16

NC = 2
NS = 16

KH = 50
K = 2 * KH
EPT2 = E2 // NS
TCH = EPT2 // KH
RPT = N // NS


def _filt_body(eba_ref, ebb_ref, w_ref, b_ref, out_ref):
    dn = (((1,), (1,)), ((), ()))
    out_ref[0, :, :DH] = lax.dot_general(
        eba_ref[...], w_ref[0], dn,
        preferred_element_type=jnp.float32) + b_ref[0]
    out_ref[0, :, DH:] = lax.dot_general(
        ebb_ref[...], w_ref[0], dn,
        preferred_element_type=jnp.float32) + b_ref[0]


def _compute_filt(edge_basis, W, b):
    BE = 8000
    nb = E2 // BE
    return pl.pallas_call(
        _filt_body,
        grid=(NC, nb),
        in_specs=[
            pl.BlockSpec((BE, DR), lambda c, i: (i, 0)),
            pl.BlockSpec((BE, DR), lambda c, i: (i + nb, 0)),
            pl.BlockSpec((1, DH, DR), lambda c, i: (c, 0, 0)),
            pl.BlockSpec((1, 1, DH), lambda c, i: (c, 0, 0)),
        ],
        out_specs=pl.BlockSpec((1, BE, D), lambda c, i: (c, i, 0)),
        out_shape=jax.ShapeDtypeStruct((NC, E2, D), jnp.float32),
    )(edge_basis, edge_basis, W.reshape(NC, DH, DR), b.reshape(NC, 1, DH))


def _sc_body(x_hbm, src_hbm, dst_hbm, filt_hbm, out_hbm, x2_hbm,
             src_v, dst_v, rows0_v, rows1_v, filt0_v, filt1_v,
             msg0_v, msg1_v, acc,
             sem_g0, sem_f0, sem_g1, sem_f1, sem_s0, sem_s1):
    cid = lax.axis_index("c")
    sid = lax.axis_index("s")

    def zrow(r, carry):
        for c in range(DH // 16):
            msg0_v[r, pl.ds(c * 16, 16)] = jnp.zeros((16,), jnp.float32)
        return carry
    lax.fori_loop(0, K, zrow, 0)
    for t in range(RPT // K):
        pltpu.sync_copy(msg0_v, acc.at[pl.ds(sid * RPT + t * K, K)])
    ztail = RPT - (RPT // K) * K
    pltpu.sync_copy(msg0_v.at[pl.ds(0, ztail)],
                    acc.at[pl.ds(sid * RPT + (RPT // K) * K, ztail)])

    r0 = sid * RPT
    pltpu.sync_copy(x_hbm.at[pl.ds(r0, RPT), pl.ds(cid * DH, DH)],
                    x2_hbm.at[cid, pl.ds(r0, RPT)])
    plsc.subcore_barrier()

    pltpu.sync_copy(src_hbm.at[sid], src_v)
    pltpu.sync_copy(dst_hbm.at[sid], dst_v)

    fbase = sid * EPT2

    def start(j, rows_v, filt_v, sem_g, sem_f):
        cp_g = pltpu.async_copy(x2_hbm.at[cid].at[src_v.at[j]], rows_v, sem_g)
        cp_f = pltpu.async_copy(
            filt_hbm.at[cid, pl.ds(fbase + j * KH, KH)], filt_v, sem_f)
        return cp_g, cp_f

    def wait_gf(rows_v, filt_v, sem_g, sem_f):
        pltpu.make_async_copy(x2_hbm.at[cid].at[src_v.at[0]],
                              rows_v, sem_g).wait()
        pltpu.make_async_copy(filt_hbm.at[cid, pl.ds(fbase, KH)],
                              filt_v, sem_f).wait()

    def wait_s(msg_v, sem_s):
        pltpu.make_async_copy(msg_v, acc.at[dst_v.at[0]], sem_s).wait()

    def finish(j, first, rows_v, filt_v, msg_v, sem_g, sem_f, sem_s):
        wait_gf(rows_v, filt_v, sem_g, sem_f)

        @pl.when(jnp.logical_not(first))
        def _():
            wait_s(msg_v, sem_s)

        def mrow(r, c2):
            for c in range(DH // 16):
                s = pl.ds(c * 16, 16)
                msg_v[r, s] = filt_v[r, s] * rows_v[r, s]
            for c in range(DH // 16):
                s = pl.ds(c * 16, 16)
                msg_v[KH + r, s] = (
                    filt_v[r, pl.ds(DH + c * 16, 16)] * rows_v[KH + r, s])
            return c2
        lax.fori_loop(0, KH, mrow, 0)

        pltpu.async_copy(msg_v, acc.at[dst_v.at[j]], sem_s, add=True)

    start(0, rows0_v, filt0_v, sem_g0, sem_f0)
    start(1, rows1_v, filt1_v, sem_g1, sem_f1)

    def pipe(j2, carry):
        a = 2 * j2
        finish(a, j2 == 0, rows0_v, filt0_v, msg0_v, sem_g0, sem_f0, sem_s0)
        start(jnp.minimum(a + 2, TCH - 1), rows0_v, filt0_v, sem_g0, sem_f0)
        finish(a + 1, j2 == 0, rows1_v, filt1_v, msg1_v,
               sem_g1, sem_f1, sem_s1)
        start(jnp.minimum(a + 3, TCH - 1), rows1_v, filt1_v, sem_g1, sem_f1)
        return carry
    lax.fori_loop(0, TCH // 2, pipe, 0)
    wait_gf(rows0_v, filt0_v, sem_g0, sem_f0)
    wait_gf(rows1_v, filt1_v, sem_g1, sem_f1)
    wait_s(msg0_v, sem_s0)
    wait_s(msg1_v, sem_s1)
    plsc.subcore_barrier()

    pltpu.sync_copy(acc.at[pl.ds(sid * RPT, RPT)],
                    out_hbm.at[pl.ds(sid * RPT, RPT), pl.ds(cid * DH, DH)])


_sc_scatter = functools.partial(
    pl.kernel,
    out_type=(
        jax.ShapeDtypeStruct((N, D), jnp.float32),
        jax.ShapeDtypeStruct((NC, N, DH), jnp.float32),
    ),
    mesh=plsc.VectorSubcoreMesh(core_axis_name="c", subcore_axis_name="s"),
    compiler_params=pltpu.CompilerParams(use_tc_tiling_on_sc=False),
    scratch_types=[
        pltpu.VMEM((TCH, K), jnp.int32),
        pltpu.VMEM((TCH, K), jnp.int32),
        pltpu.VMEM((K, DH), jnp.float32),
        pltpu.VMEM((K, DH), jnp.float32),
        pltpu.VMEM((KH, D), jnp.float32),
        pltpu.VMEM((KH, D), jnp.float32),
        pltpu.VMEM((K, DH), jnp.float32),
        pltpu.VMEM((K, DH), jnp.float32),
        pltpu.VMEM_SHARED((N, DH), jnp.float32),
        pltpu.SemaphoreType.DMA,
        pltpu.SemaphoreType.DMA,
        pltpu.SemaphoreType.DMA,
        pltpu.SemaphoreType.DMA,
        pltpu.SemaphoreType.DMA,
        pltpu.SemaphoreType.DMA,
    ],
)(_sc_body)


def kernel(x, edge_index, edge_basis, W, b):
    filt2 = _compute_filt(edge_basis, W, b)
    src = edge_index[0]
    dst = edge_index[1]
    srcA = src[:E2].reshape(NS, TCH, KH)
    srcB = src[E2:].reshape(NS, TCH, KH)
    src_arr = jnp.concatenate([srcA, srcB], axis=2)
    dstA = dst[:E2].reshape(NS, TCH, KH)
    dstB = dst[E2:].reshape(NS, TCH, KH)
    dst_arr = jnp.concatenate([dstA, dstB], axis=2)
    out, _ = _sc_scatter(x, src_arr, dst_arr, filt2)
    return out


# speedup vs baseline: 1.2969x; 1.2969x over previous
import functools

import jax
import jax.numpy as jnp
from jax import lax
from jax.experimental import pallas as pl
from jax.experimental.pallas import tpu as pltpu
from jax.experimental.pallas import tpu_sc as plsc

N = 10000
E = 320000
E2 = E // 2
D = 128
DH = D // 2
DR = 16

NC = 2
NS = 16

_DIAG = 0
KH = 50
K = 2 * KH
EPT2 = E2 // NS
TCH = EPT2 // KH
RPT = N // NS


def _filt_body(eba_ref, ebb_ref, w_ref, b_ref, out_ref):
    dn = (((1,), (1,)), ((), ()))
    out_ref[0, :, :DH] = lax.dot_general(
        eba_ref[...], w_ref[0], dn,
        preferred_element_type=jnp.float32) + b_ref[0]
    out_ref[0, :, DH:] = lax.dot_general(
        ebb_ref[...], w_ref[0], dn,
        preferred_element_type=jnp.float32) + b_ref[0]


def _compute_filt(edge_basis, W, b):
    BE = 8000
    nb = E2 // BE
    return pl.pallas_call(
        _filt_body,
        grid=(NC, nb),
        in_specs=[
            pl.BlockSpec((BE, DR), lambda c, i: (i, 0)),
            pl.BlockSpec((BE, DR), lambda c, i: (i + nb, 0)),
            pl.BlockSpec((1, DH, DR), lambda c, i: (c, 0, 0)),
            pl.BlockSpec((1, 1, DH), lambda c, i: (c, 0, 0)),
        ],
        out_specs=pl.BlockSpec((1, BE, D), lambda c, i: (c, i, 0)),
        out_shape=jax.ShapeDtypeStruct((NC, E2, D), jnp.float32),
    )(edge_basis, edge_basis, W.reshape(NC, DH, DR), b.reshape(NC, 1, DH))


def _sc_body(x_hbm, src_hbm, dst_hbm, filt_hbm, out_hbm, x2_hbm,
             src_v, dst_v, rows0_v, rows1_v, filt0_v, filt1_v,
             msg0_v, msg1_v, acc,
             sem_g0, sem_f0, sem_g1, sem_f1, sem_s0, sem_s1):
    cid = lax.axis_index("c")
    sid = lax.axis_index("s")

    def zrow(r, carry):
        for c in range(DH // 16):
            msg0_v[r, pl.ds(c * 16, 16)] = jnp.zeros((16,), jnp.float32)
        return carry
    lax.fori_loop(0, K, zrow, 0)
    for t in range(RPT // K):
        pltpu.sync_copy(msg0_v, acc.at[pl.ds(sid * RPT + t * K, K)])
    ztail = RPT - (RPT // K) * K
    pltpu.sync_copy(msg0_v.at[pl.ds(0, ztail)],
                    acc.at[pl.ds(sid * RPT + (RPT // K) * K, ztail)])

    r0 = sid * RPT
    pltpu.sync_copy(x_hbm.at[pl.ds(r0, RPT), pl.ds(cid * DH, DH)],
                    x2_hbm.at[cid, pl.ds(r0, RPT)])
    plsc.subcore_barrier()

    pltpu.sync_copy(src_hbm.at[sid], src_v)
    pltpu.sync_copy(dst_hbm.at[sid], dst_v)

    fbase = sid * EPT2

    def start(j, rows_v, filt_v, sem_g, sem_f):
        cp_g = pltpu.async_copy(x2_hbm.at[cid].at[src_v.at[j]], rows_v, sem_g)
        cp_f = pltpu.async_copy(
            filt_hbm.at[cid, pl.ds(fbase + j * KH, KH)], filt_v, sem_f)
        return cp_g, cp_f

    def wait_gf(rows_v, filt_v, sem_g, sem_f):
        pltpu.make_async_copy(x2_hbm.at[cid].at[src_v.at[0]],
                              rows_v, sem_g).wait()
        pltpu.make_async_copy(filt_hbm.at[cid, pl.ds(fbase, KH)],
                              filt_v, sem_f).wait()

    def wait_s(msg_v, sem_s):
        pltpu.make_async_copy(msg_v, acc.at[dst_v.at[0]], sem_s).wait()

    def finish(j, first, rows_v, filt_v, msg_v, sem_g, sem_f, sem_s):
        wait_gf(rows_v, filt_v, sem_g, sem_f)

        if _DIAG != 2:
            @pl.when(jnp.logical_not(first))
            def _():
                wait_s(msg_v, sem_s)

        if _DIAG != 1:
            @plsc.parallel_loop(0, KH, unroll=4)
            def mrow(r):
                for c in range(DH // 16):
                    s = pl.ds(c * 16, 16)
                    msg_v[r, s] = filt_v[r, s] * rows_v[r, s]
                for c in range(DH // 16):
                    s = pl.ds(c * 16, 16)
                    msg_v[KH + r, s] = (
                        filt_v[r, pl.ds(DH + c * 16, 16)] * rows_v[KH + r, s])

        if _DIAG != 2:
            pltpu.async_copy(msg_v, acc.at[dst_v.at[j]], sem_s, add=True)

    start(0, rows0_v, filt0_v, sem_g0, sem_f0)
    start(1, rows1_v, filt1_v, sem_g1, sem_f1)

    def pipe(j2, carry):
        a = 2 * j2
        finish(a, j2 == 0, rows0_v, filt0_v, msg0_v, sem_g0, sem_f0, sem_s0)
        start(jnp.minimum(a + 2, TCH - 1), rows0_v, filt0_v, sem_g0, sem_f0)
        finish(a + 1, j2 == 0, rows1_v, filt1_v, msg1_v,
               sem_g1, sem_f1, sem_s1)
        start(jnp.minimum(a + 3, TCH - 1), rows1_v, filt1_v, sem_g1, sem_f1)
        return carry
    lax.fori_loop(0, TCH // 2, pipe, 0)
    wait_gf(rows0_v, filt0_v, sem_g0, sem_f0)
    wait_gf(rows1_v, filt1_v, sem_g1, sem_f1)
    if _DIAG != 2:
        wait_s(msg0_v, sem_s0)
        wait_s(msg1_v, sem_s1)
    plsc.subcore_barrier()

    pltpu.sync_copy(acc.at[pl.ds(sid * RPT, RPT)],
                    out_hbm.at[pl.ds(sid * RPT, RPT), pl.ds(cid * DH, DH)])


_sc_scatter = functools.partial(
    pl.kernel,
    out_type=(
        jax.ShapeDtypeStruct((N, D), jnp.float32),
        jax.ShapeDtypeStruct((NC, N, DH), jnp.float32),
    ),
    mesh=plsc.VectorSubcoreMesh(core_axis_name="c", subcore_axis_name="s"),
    compiler_params=pltpu.CompilerParams(use_tc_tiling_on_sc=False),
    scratch_types=[
        pltpu.VMEM((TCH, K), jnp.int32),
        pltpu.VMEM((TCH, K), jnp.int32),
        pltpu.VMEM((K, DH), jnp.float32),
        pltpu.VMEM((K, DH), jnp.float32),
        pltpu.VMEM((KH, D), jnp.float32),
        pltpu.VMEM((KH, D), jnp.float32),
        pltpu.VMEM((K, DH), jnp.float32),
        pltpu.VMEM((K, DH), jnp.float32),
        pltpu.VMEM_SHARED((N, DH), jnp.float32),
        pltpu.SemaphoreType.DMA,
        pltpu.SemaphoreType.DMA,
        pltpu.SemaphoreType.DMA,
        pltpu.SemaphoreType.DMA,
        pltpu.SemaphoreType.DMA,
        pltpu.SemaphoreType.DMA,
    ],
)(_sc_body)


def kernel(x, edge_index, edge_basis, W, b):
    filt2 = _compute_filt(edge_basis, W, b)
    src = edge_index[0]
    dst = edge_index[1]
    srcA = src[:E2].reshape(NS, TCH, KH)
    srcB = src[E2:].reshape(NS, TCH, KH)
    src_arr = jnp.concatenate([srcA, srcB], axis=2)
    dstA = dst[:E2].reshape(NS, TCH, KH)
    dstB = dst[E2:].reshape(NS, TCH, KH)
    dst_arr = jnp.concatenate([dstA, dstB], axis=2)
    out, _ = _sc_scatter(x, src_arr, dst_arr, filt2)
    return out


# speedup vs baseline: 1.3916x; 1.0730x over previous
import functools

import jax
import jax.numpy as jnp
from jax import lax
from jax.experimental import pallas as pl
from jax.experimental.pallas import tpu as pltpu
from jax.experimental.pallas import tpu_sc as plsc

N = 10000
E = 320000
E2 = E // 2
D = 128
DH = D // 2
DR = 16

NC = 2
NS = 16

_DIAG = 0
KH = 50
K = 2 * KH
EPT2 = E2 // NS
TCH = EPT2 // KH
RPT = N // NS


_BE = 4000
_NB = E2 // _BE


def _filt_body(eb_hbm, w_ref, b_ref, out_ref, a0, b0, a1, b1, s0, s1):
    i = pl.program_id(0)
    dn = (((1,), (1,)), ((), ()))

    def fetch(blk, bufa, bufb, sem):
        pltpu.async_copy(eb_hbm.at[pl.ds(blk * _BE, _BE)], bufa, sem)
        pltpu.async_copy(eb_hbm.at[pl.ds(E2 + blk * _BE, _BE)], bufb, sem)

    def wait(bufa, bufb, sem):
        pltpu.make_async_copy(eb_hbm.at[pl.ds(0, _BE)], bufa, sem).wait()
        pltpu.make_async_copy(eb_hbm.at[pl.ds(0, _BE)], bufb, sem).wait()

    def compute(bufa, bufb, sem):
        wait(bufa, bufb, sem)
        eba = bufa[...]
        ebb = bufb[...]
        for c in range(NC):
            out_ref[c, :, :DH] = lax.dot_general(
                eba, w_ref[c], dn,
                preferred_element_type=jnp.float32) + b_ref[c]
            out_ref[c, :, DH:] = lax.dot_general(
                ebb, w_ref[c], dn,
                preferred_element_type=jnp.float32) + b_ref[c]

    @pl.when(i == 0)
    def _prime():
        fetch(0, a0, b0, s0)

    nxt = jnp.minimum(i + 1, _NB - 1)

    @pl.when(i % 2 == 0)
    def _even():
        fetch(nxt, a1, b1, s1)
        compute(a0, b0, s0)

    @pl.when(i % 2 == 1)
    def _odd():
        fetch(nxt, a0, b0, s0)
        compute(a1, b1, s1)

    @pl.when(i == _NB - 1)
    def _drain():
        @pl.when(i % 2 == 0)
        def _():
            wait(a1, b1, s1)

        @pl.when(i % 2 == 1)
        def _():
            wait(a0, b0, s0)


def _compute_filt(edge_basis, W, b):
    return pl.pallas_call(
        _filt_body,
        grid=(_NB,),
        in_specs=[
            pl.BlockSpec(memory_space=pltpu.MemorySpace.HBM),
            pl.BlockSpec((NC, DH, DR), lambda i: (0, 0, 0)),
            pl.BlockSpec((NC, 1, DH), lambda i: (0, 0, 0)),
        ],
        out_specs=pl.BlockSpec((NC, _BE, D), lambda i: (0, i, 0)),
        out_shape=jax.ShapeDtypeStruct((NC, E2, D), jnp.float32),
        scratch_shapes=[
            pltpu.VMEM((_BE, DR), jnp.float32),
            pltpu.VMEM((_BE, DR), jnp.float32),
            pltpu.VMEM((_BE, DR), jnp.float32),
            pltpu.VMEM((_BE, DR), jnp.float32),
            pltpu.SemaphoreType.DMA,
            pltpu.SemaphoreType.DMA,
        ],
    )(edge_basis, W.reshape(NC, DH, DR), b.reshape(NC, 1, DH))


def _sc_body(x_hbm, src_hbm, dst_hbm, filt_hbm, out_hbm, x2_hbm,
             src_v, dst_v, rows0_v, rows1_v, filt0_v, filt1_v,
             msg0_v, msg1_v, acc,
             sem_g0, sem_f0, sem_g1, sem_f1, sem_s0, sem_s1):
    cid = lax.axis_index("c")
    sid = lax.axis_index("s")

    def zrow(r, carry):
        for c in range(DH // 16):
            msg0_v[r, pl.ds(c * 16, 16)] = jnp.zeros((16,), jnp.float32)
        return carry
    lax.fori_loop(0, K, zrow, 0)
    for t in range(RPT // K):
        pltpu.sync_copy(msg0_v, acc.at[pl.ds(sid * RPT + t * K, K)])
    ztail = RPT - (RPT // K) * K
    pltpu.sync_copy(msg0_v.at[pl.ds(0, ztail)],
                    acc.at[pl.ds(sid * RPT + (RPT // K) * K, ztail)])

    r0 = sid * RPT
    pltpu.sync_copy(x_hbm.at[pl.ds(r0, RPT), pl.ds(cid * DH, DH)],
                    x2_hbm.at[cid, pl.ds(r0, RPT)])
    plsc.subcore_barrier()

    pltpu.sync_copy(src_hbm.at[sid], src_v)
    pltpu.sync_copy(dst_hbm.at[sid], dst_v)

    fbase = sid * EPT2

    def start(j, rows_v, filt_v, sem_g, sem_f):
        cp_g = pltpu.async_copy(x2_hbm.at[cid].at[src_v.at[j]], rows_v, sem_g)
        cp_f = pltpu.async_copy(
            filt_hbm.at[cid, pl.ds(fbase + j * KH, KH)], filt_v, sem_f)
        return cp_g, cp_f

    def wait_gf(rows_v, filt_v, sem_g, sem_f):
        pltpu.make_async_copy(x2_hbm.at[cid].at[src_v.at[0]],
                              rows_v, sem_g).wait()
        pltpu.make_async_copy(filt_hbm.at[cid, pl.ds(fbase, KH)],
                              filt_v, sem_f).wait()

    def wait_s(msg_v, sem_s):
        pltpu.make_async_copy(msg_v, acc.at[dst_v.at[0]], sem_s).wait()

    def finish(j, first, rows_v, filt_v, msg_v, sem_g, sem_f, sem_s):
        wait_gf(rows_v, filt_v, sem_g, sem_f)

        if _DIAG != 2:
            @pl.when(jnp.logical_not(first))
            def _():
                wait_s(msg_v, sem_s)

        if _DIAG != 1:
            @plsc.parallel_loop(0, KH, unroll=4)
            def mrow(r):
                for c in range(DH // 16):
                    s = pl.ds(c * 16, 16)
                    msg_v[r, s] = filt_v[r, s] * rows_v[r, s]
                for c in range(DH // 16):
                    s = pl.ds(c * 16, 16)
                    msg_v[KH + r, s] = (
                        filt_v[r, pl.ds(DH + c * 16, 16)] * rows_v[KH + r, s])

        if _DIAG != 2:
            pltpu.async_copy(msg_v, acc.at[dst_v.at[j]], sem_s, add=True)

    start(0, rows0_v, filt0_v, sem_g0, sem_f0)
    start(1, rows1_v, filt1_v, sem_g1, sem_f1)

    def pipe(j2, carry):
        a = 2 * j2
        finish(a, j2 == 0, rows0_v, filt0_v, msg0_v, sem_g0, sem_f0, sem_s0)
        start(jnp.minimum(a + 2, TCH - 1), rows0_v, filt0_v, sem_g0, sem_f0)
        finish(a + 1, j2 == 0, rows1_v, filt1_v, msg1_v,
               sem_g1, sem_f1, sem_s1)
        start(jnp.minimum(a + 3, TCH - 1), rows1_v, filt1_v, sem_g1, sem_f1)
        return carry
    lax.fori_loop(0, TCH // 2, pipe, 0)
    wait_gf(rows0_v, filt0_v, sem_g0, sem_f0)
    wait_gf(rows1_v, filt1_v, sem_g1, sem_f1)
    if _DIAG != 2:
        wait_s(msg0_v, sem_s0)
        wait_s(msg1_v, sem_s1)
    plsc.subcore_barrier()

    pltpu.sync_copy(acc.at[pl.ds(sid * RPT, RPT)],
                    out_hbm.at[pl.ds(sid * RPT, RPT), pl.ds(cid * DH, DH)])


_sc_scatter = functools.partial(
    pl.kernel,
    out_type=(
        jax.ShapeDtypeStruct((N, D), jnp.float32),
        jax.ShapeDtypeStruct((NC, N, DH), jnp.float32),
    ),
    mesh=plsc.VectorSubcoreMesh(core_axis_name="c", subcore_axis_name="s"),
    compiler_params=pltpu.CompilerParams(use_tc_tiling_on_sc=False),
    scratch_types=[
        pltpu.VMEM((TCH, K), jnp.int32),
        pltpu.VMEM((TCH, K), jnp.int32),
        pltpu.VMEM((K, DH), jnp.float32),
        pltpu.VMEM((K, DH), jnp.float32),
        pltpu.VMEM((KH, D), jnp.float32),
        pltpu.VMEM((KH, D), jnp.float32),
        pltpu.VMEM((K, DH), jnp.float32),
        pltpu.VMEM((K, DH), jnp.float32),
        pltpu.VMEM_SHARED((N, DH), jnp.float32),
        pltpu.SemaphoreType.DMA,
        pltpu.SemaphoreType.DMA,
        pltpu.SemaphoreType.DMA,
        pltpu.SemaphoreType.DMA,
        pltpu.SemaphoreType.DMA,
        pltpu.SemaphoreType.DMA,
    ],
)(_sc_body)


def kernel(x, edge_index, edge_basis, W, b):
    filt2 = _compute_filt(edge_basis, W, b)
    src = edge_index[0]
    dst = edge_index[1]
    srcA = src[:E2].reshape(NS, TCH, KH)
    srcB = src[E2:].reshape(NS, TCH, KH)
    src_arr = jnp.concatenate([srcA, srcB], axis=2)
    dstA = dst[:E2].reshape(NS, TCH, KH)
    dstB = dst[E2:].reshape(NS, TCH, KH)
    dst_arr = jnp.concatenate([dstA, dstB], axis=2)
    out, _ = _sc_scatter(x, src_arr, dst_arr, filt2)
    return out


# speedup vs baseline: 1.6649x; 1.1964x over previous
import functools

import jax
import jax.numpy as jnp
from jax import lax
from jax.experimental import pallas as pl
from jax.experimental.pallas import tpu as pltpu
from jax.experimental.pallas import tpu_sc as plsc

N = 10000
E = 320000
E2 = E // 2
D = 128
DH = D // 2
DR = 16

NC = 2
NS = 16

_DIAG = 0
KH = 50
K = 2 * KH
EPT2 = E2 // NS
TCH = EPT2 // KH
RPT = N // NS


_BE = 6400
_NB = E2 // _BE


def _filt_body(ebta_ref, ebtb_ref, w_ref, b_ref, out_ref):
    dn = (((0,), (1,)), ((), ()))
    ebta = ebta_ref[...]
    ebtb = ebtb_ref[...]
    for c in range(NC):
        out_ref[c, :, :DH] = lax.dot_general(
            ebta, w_ref[c], dn, preferred_element_type=jnp.float32) + b_ref[c]
        out_ref[c, :, DH:] = lax.dot_general(
            ebtb, w_ref[c], dn, preferred_element_type=jnp.float32) + b_ref[c]


def _compute_filt(edge_basis, W, b):
    return pl.pallas_call(
        _filt_body,
        grid=(_NB,),
        in_specs=[
            pl.BlockSpec((DR, _BE), lambda i: (0, i)),
            pl.BlockSpec((DR, _BE), lambda i: (0, i + _NB)),
            pl.BlockSpec((NC, DH, DR), lambda i: (0, 0, 0)),
            pl.BlockSpec((NC, 1, DH), lambda i: (0, 0, 0)),
        ],
        out_specs=pl.BlockSpec((NC, _BE, D), lambda i: (0, i, 0)),
        out_shape=jax.ShapeDtypeStruct((NC, E2, D), jnp.float32),
    )(edge_basis.T, edge_basis.T, W.reshape(NC, DH, DR), b.reshape(NC, 1, DH))


def _sc_body(x_hbm, src_hbm, dst_hbm, filt_hbm, out_hbm, x2_hbm,
             src_v, dst_v, rows0_v, rows1_v, filt0_v, filt1_v,
             msg0_v, msg1_v, acc,
             sem_g0, sem_f0, sem_g1, sem_f1, sem_s0, sem_s1):
    cid = lax.axis_index("c")
    sid = lax.axis_index("s")

    def zrow(r, carry):
        for c in range(DH // 16):
            msg0_v[r, pl.ds(c * 16, 16)] = jnp.zeros((16,), jnp.float32)
        return carry
    lax.fori_loop(0, K, zrow, 0)
    for t in range(RPT // K):
        pltpu.sync_copy(msg0_v, acc.at[pl.ds(sid * RPT + t * K, K)])
    ztail = RPT - (RPT // K) * K
    pltpu.sync_copy(msg0_v.at[pl.ds(0, ztail)],
                    acc.at[pl.ds(sid * RPT + (RPT // K) * K, ztail)])

    r0 = sid * RPT
    pltpu.sync_copy(x_hbm.at[pl.ds(r0, RPT), pl.ds(cid * DH, DH)],
                    x2_hbm.at[cid, pl.ds(r0, RPT)])
    plsc.subcore_barrier()

    pltpu.sync_copy(src_hbm.at[sid], src_v)
    pltpu.sync_copy(dst_hbm.at[sid], dst_v)

    fbase = sid * EPT2

    def start(j, rows_v, filt_v, sem_g, sem_f):
        cp_g = pltpu.async_copy(x2_hbm.at[cid].at[src_v.at[j]], rows_v, sem_g)
        cp_f = pltpu.async_copy(
            filt_hbm.at[cid, pl.ds(fbase + j * KH, KH)], filt_v, sem_f)
        return cp_g, cp_f

    def wait_gf(rows_v, filt_v, sem_g, sem_f):
        pltpu.make_async_copy(x2_hbm.at[cid].at[src_v.at[0]],
                              rows_v, sem_g).wait()
        pltpu.make_async_copy(filt_hbm.at[cid, pl.ds(fbase, KH)],
                              filt_v, sem_f).wait()

    def wait_s(msg_v, sem_s):
        pltpu.make_async_copy(msg_v, acc.at[dst_v.at[0]], sem_s).wait()

    def finish(j, first, rows_v, filt_v, msg_v, sem_g, sem_f, sem_s):
        wait_gf(rows_v, filt_v, sem_g, sem_f)

        if _DIAG != 2:
            @pl.when(jnp.logical_not(first))
            def _():
                wait_s(msg_v, sem_s)

        if _DIAG != 1:
            @plsc.parallel_loop(0, KH, unroll=4)
            def mrow(r):
                for c in range(DH // 16):
                    s = pl.ds(c * 16, 16)
                    msg_v[r, s] = filt_v[r, s] * rows_v[r, s]
                for c in range(DH // 16):
                    s = pl.ds(c * 16, 16)
                    msg_v[KH + r, s] = (
                        filt_v[r, pl.ds(DH + c * 16, 16)] * rows_v[KH + r, s])

        if _DIAG != 2:
            pltpu.async_copy(msg_v, acc.at[dst_v.at[j]], sem_s, add=True)

    start(0, rows0_v, filt0_v, sem_g0, sem_f0)
    start(1, rows1_v, filt1_v, sem_g1, sem_f1)

    def pipe(j2, carry):
        a = 2 * j2
        finish(a, j2 == 0, rows0_v, filt0_v, msg0_v, sem_g0, sem_f0, sem_s0)
        start(jnp.minimum(a + 2, TCH - 1), rows0_v, filt0_v, sem_g0, sem_f0)
        finish(a + 1, j2 == 0, rows1_v, filt1_v, msg1_v,
               sem_g1, sem_f1, sem_s1)
        start(jnp.minimum(a + 3, TCH - 1), rows1_v, filt1_v, sem_g1, sem_f1)
        return carry
    lax.fori_loop(0, TCH // 2, pipe, 0)
    wait_gf(rows0_v, filt0_v, sem_g0, sem_f0)
    wait_gf(rows1_v, filt1_v, sem_g1, sem_f1)
    if _DIAG != 2:
        wait_s(msg0_v, sem_s0)
        wait_s(msg1_v, sem_s1)
    plsc.subcore_barrier()

    pltpu.sync_copy(acc.at[pl.ds(sid * RPT, RPT)],
                    out_hbm.at[pl.ds(sid * RPT, RPT), pl.ds(cid * DH, DH)])


_sc_scatter = functools.partial(
    pl.kernel,
    out_type=(
        jax.ShapeDtypeStruct((N, D), jnp.float32),
        jax.ShapeDtypeStruct((NC, N, DH), jnp.float32),
    ),
    mesh=plsc.VectorSubcoreMesh(core_axis_name="c", subcore_axis_name="s"),
    compiler_params=pltpu.CompilerParams(use_tc_tiling_on_sc=False),
    scratch_types=[
        pltpu.VMEM((TCH, K), jnp.int32),
        pltpu.VMEM((TCH, K), jnp.int32),
        pltpu.VMEM((K, DH), jnp.float32),
        pltpu.VMEM((K, DH), jnp.float32),
        pltpu.VMEM((KH, D), jnp.float32),
        pltpu.VMEM((KH, D), jnp.float32),
        pltpu.VMEM((K, DH), jnp.float32),
        pltpu.VMEM((K, DH), jnp.float32),
        pltpu.VMEM_SHARED((N, DH), jnp.float32),
        pltpu.SemaphoreType.DMA,
        pltpu.SemaphoreType.DMA,
        pltpu.SemaphoreType.DMA,
        pltpu.SemaphoreType.DMA,
        pltpu.SemaphoreType.DMA,
        pltpu.SemaphoreType.DMA,
    ],
)(_sc_body)


def kernel(x, edge_index, edge_basis, W, b):
    filt2 = _compute_filt(edge_basis, W, b)
    src = edge_index[0]
    dst = edge_index[1]
    srcA = src[:E2].reshape(NS, TCH, KH)
    srcB = src[E2:].reshape(NS, TCH, KH)
    src_arr = jnp.concatenate([srcA, srcB], axis=2)
    dstA = dst[:E2].reshape(NS, TCH, KH)
    dstB = dst[E2:].reshape(NS, TCH, KH)
    dst_arr = jnp.concatenate([dstA, dstB], axis=2)
    out, _ = _sc_scatter(x, src_arr, dst_arr, filt2)
    return out


# speedup vs baseline: 2.4424x; 1.4669x over previous
import functools

import jax
import jax.numpy as jnp
from jax import lax
from jax.experimental import pallas as pl
from jax.experimental.pallas import tpu as pltpu
from jax.experimental.pallas import tpu_sc as plsc

N = 10000
E = 320000
E2 = E // 2
D = 128
DH = D // 2
DR = 16

NC = 2
NS = 16

_DIAG = 0
KH = 50
K = 2 * KH
EPT2 = E2 // NS
TCH = EPT2 // KH
NPASS = 4
TPP = TCH // NPASS
RPT = N // NS


_BE = 6400
_NB = E2 // _BE


def _filt_body(ebta_ref, ebtb_ref, w_ref, b_ref, out_ref):
    dn = (((0,), (1,)), ((), ()))
    ebta = ebta_ref[...]
    ebtb = ebtb_ref[...]
    for c in range(NC):
        out_ref[c, :, :DH] = lax.dot_general(
            ebta, w_ref[c], dn, preferred_element_type=jnp.float32) + b_ref[c]
        out_ref[c, :, DH:] = lax.dot_general(
            ebtb, w_ref[c], dn, preferred_element_type=jnp.float32) + b_ref[c]


def _compute_filt(edge_basis, W, b):
    return pl.pallas_call(
        _filt_body,
        grid=(_NB,),
        in_specs=[
            pl.BlockSpec((DR, _BE), lambda i: (0, i)),
            pl.BlockSpec((DR, _BE), lambda i: (0, i + _NB)),
            pl.BlockSpec((NC, DH, DR), lambda i: (0, 0, 0)),
            pl.BlockSpec((NC, 1, DH), lambda i: (0, 0, 0)),
        ],
        out_specs=pl.BlockSpec((NC, _BE, D), lambda i: (0, i, 0)),
        out_shape=jax.ShapeDtypeStruct((NC, E2, D), jnp.float32),
    )(edge_basis.T, edge_basis.T, W.reshape(NC, DH, DR), b.reshape(NC, 1, DH))


def _sc_body(x_hbm, src_hbm, dst_hbm, filt_hbm, out_hbm,
             src_v, dst_v, rows0_v, rows1_v, filt0_v, filt1_v,
             msg0_v, msg1_v, xtab, acc,
             sem_g0, sem_f0, sem_g1, sem_f1, sem_s0, sem_s1):
    cid = lax.axis_index("c")
    sid = lax.axis_index("s")

    def zrow(r, carry):
        for c in range(DH // 16):
            msg0_v[r, pl.ds(c * 16, 16)] = jnp.zeros((16,), jnp.float32)
        return carry
    lax.fori_loop(0, K, zrow, 0)
    for t in range(RPT // K):
        pltpu.sync_copy(msg0_v, acc.at[pl.ds(sid * RPT + t * K, K)])
    ztail = RPT - (RPT // K) * K
    pltpu.sync_copy(msg0_v.at[pl.ds(0, ztail)],
                    acc.at[pl.ds(sid * RPT + (RPT // K) * K, ztail)])

    r0 = sid * RPT
    pltpu.sync_copy(x_hbm.at[pl.ds(r0, RPT), pl.ds(cid * DH, DH)],
                    xtab.at[pl.ds(r0, RPT)])
    plsc.subcore_barrier()

    fbase = sid * EPT2

    def start(jb, jl, rows_v, filt_v, sem_g, sem_f):
        cp_g = pltpu.async_copy(xtab.at[src_v.at[jl]], rows_v, sem_g)
        cp_f = pltpu.async_copy(
            filt_hbm.at[cid, pl.ds(fbase + (jb + jl) * KH, KH)],
            filt_v, sem_f)
        return cp_g, cp_f

    def wait_gf(rows_v, filt_v, sem_g, sem_f):
        pltpu.make_async_copy(xtab.at[src_v.at[0]],
                              rows_v, sem_g).wait()
        pltpu.make_async_copy(filt_hbm.at[cid, pl.ds(fbase, KH)],
                              filt_v, sem_f).wait()

    def wait_s(msg_v, sem_s):
        pltpu.make_async_copy(msg_v, acc.at[dst_v.at[0]], sem_s).wait()

    def finish(jb, jl, first, rows_v, filt_v, msg_v, sem_g, sem_f, sem_s):
        wait_gf(rows_v, filt_v, sem_g, sem_f)

        if _DIAG != 2:
            @pl.when(jnp.logical_not(first))
            def _():
                wait_s(msg_v, sem_s)

        if _DIAG != 1:
            @plsc.parallel_loop(0, KH, unroll=4)
            def mrow(r):
                for c in range(DH // 16):
                    s = pl.ds(c * 16, 16)
                    msg_v[r, s] = filt_v[r, s] * rows_v[r, s]
                for c in range(DH // 16):
                    s = pl.ds(c * 16, 16)
                    msg_v[KH + r, s] = (
                        filt_v[r, pl.ds(DH + c * 16, 16)] * rows_v[KH + r, s])

        if _DIAG != 2:
            pltpu.async_copy(msg_v, acc.at[dst_v.at[jl]], sem_s, add=True)

    for p in range(NPASS):
        jb = p * TPP
        pltpu.sync_copy(src_hbm.at[sid, pl.ds(jb, TPP)], src_v)
        pltpu.sync_copy(dst_hbm.at[sid, pl.ds(jb, TPP)], dst_v)
        start(jb, 0, rows0_v, filt0_v, sem_g0, sem_f0)
        start(jb, 1, rows1_v, filt1_v, sem_g1, sem_f1)

        def pipe(j2, carry):
            a = 2 * j2
            finish(jb, a, j2 == 0, rows0_v, filt0_v, msg0_v,
                   sem_g0, sem_f0, sem_s0)
            start(jb, jnp.minimum(a + 2, TPP - 1), rows0_v, filt0_v,
                  sem_g0, sem_f0)
            finish(jb, a + 1, j2 == 0, rows1_v, filt1_v, msg1_v,
                   sem_g1, sem_f1, sem_s1)
            start(jb, jnp.minimum(a + 3, TPP - 1), rows1_v, filt1_v,
                  sem_g1, sem_f1)
            return carry
        lax.fori_loop(0, TPP // 2, pipe, 0)
        wait_gf(rows0_v, filt0_v, sem_g0, sem_f0)
        wait_gf(rows1_v, filt1_v, sem_g1, sem_f1)
        if _DIAG != 2:
            wait_s(msg0_v, sem_s0)
            wait_s(msg1_v, sem_s1)
    plsc.subcore_barrier()

    pltpu.sync_copy(acc.at[pl.ds(sid * RPT, RPT)],
                    out_hbm.at[pl.ds(sid * RPT, RPT), pl.ds(cid * DH, DH)])


_sc_scatter = functools.partial(
    pl.kernel,
    out_type=jax.ShapeDtypeStruct((N, D), jnp.float32),
    mesh=plsc.VectorSubcoreMesh(core_axis_name="c", subcore_axis_name="s"),
    compiler_params=pltpu.CompilerParams(use_tc_tiling_on_sc=False),
    scratch_types=[
        pltpu.VMEM((TPP, K), jnp.int32),
        pltpu.VMEM((TPP, K), jnp.int32),
        pltpu.VMEM((K, DH), jnp.float32),
        pltpu.VMEM((K, DH), jnp.float32),
        pltpu.VMEM((KH, D), jnp.float32),
        pltpu.VMEM((KH, D), jnp.float32),
        pltpu.VMEM((K, DH), jnp.float32),
        pltpu.VMEM((K, DH), jnp.float32),
        pltpu.VMEM_SHARED((N, DH), jnp.float32),
        pltpu.VMEM_SHARED((N, DH), jnp.float32),
        pltpu.SemaphoreType.DMA,
        pltpu.SemaphoreType.DMA,
        pltpu.SemaphoreType.DMA,
        pltpu.SemaphoreType.DMA,
        pltpu.SemaphoreType.DMA,
        pltpu.SemaphoreType.DMA,
    ],
)(_sc_body)


def kernel(x, edge_index, edge_basis, W, b):
    filt2 = _compute_filt(edge_basis, W, b)
    src = edge_index[0]
    dst = edge_index[1]
    srcA = src[:E2].reshape(NS, TCH, KH)
    srcB = src[E2:].reshape(NS, TCH, KH)
    src_arr = jnp.concatenate([srcA, srcB], axis=2)
    dstA = dst[:E2].reshape(NS, TCH, KH)
    dstB = dst[E2:].reshape(NS, TCH, KH)
    dst_arr = jnp.concatenate([dstA, dstB], axis=2)
    return _sc_scatter(x, src_arr, dst_arr, filt2)


# speedup vs baseline: 2.7062x; 1.1080x over previous
import functools

import jax
import jax.numpy as jnp
from jax import lax
from jax.experimental import pallas as pl
from jax.experimental.pallas import tpu as pltpu
from jax.experimental.pallas import tpu_sc as plsc

N = 10000
E = 320000
E2 = E // 2
D = 128
DH = D // 2
DR = 16

NC = 2
NS = 16

KH = 40
K = 2 * KH
EPT2 = E2 // NS
TCH = EPT2 // KH
NPASS = 5
TPP = TCH // NPASS
IPP = TPP * KH
RPT = N // NS


_BE = 6400
_NB = E2 // _BE


def _filt_body(ebta_ref, ebtb_ref, w_ref, b_ref, out_ref):
    dn = (((0,), (1,)), ((), ()))
    ebta = ebta_ref[...]
    ebtb = ebtb_ref[...]
    for c in range(NC):
        out_ref[c, :, :DH] = lax.dot_general(
            ebta, w_ref[c], dn, preferred_element_type=jnp.float32) + b_ref[c]
        out_ref[c, :, DH:] = lax.dot_general(
            ebtb, w_ref[c], dn, preferred_element_type=jnp.float32) + b_ref[c]


def _compute_filt(edge_basis, W, b):
    return pl.pallas_call(
        _filt_body,
        grid=(_NB,),
        in_specs=[
            pl.BlockSpec((DR, _BE), lambda i: (0, i)),
            pl.BlockSpec((DR, _BE), lambda i: (0, i + _NB)),
            pl.BlockSpec((NC, DH, DR), lambda i: (0, 0, 0)),
            pl.BlockSpec((NC, 1, DH), lambda i: (0, 0, 0)),
        ],
        out_specs=pl.BlockSpec((NC, _BE, D), lambda i: (0, i, 0)),
        out_shape=jax.ShapeDtypeStruct((NC, E2, D), jnp.float32),
    )(edge_basis.T, edge_basis.T, W.reshape(NC, DH, DR), b.reshape(NC, 1, DH))


def _sc_body(x_hbm, ei_hbm, filt_hbm, out_hbm,
             srcl_v, srch_v, dstl_v, dsth_v,
             rows0_v, rows1_v, filt0_v, filt1_v,
             msg0_v, msg1_v, xtab, acc,
             sem_g0, sem_f0, sem_g1, sem_f1, sem_s0, sem_s1):
    cid = lax.axis_index("c")
    sid = lax.axis_index("s")

    def zrow(r, carry):
        for c in range(DH // 16):
            msg0_v[r, pl.ds(c * 16, 16)] = jnp.zeros((16,), jnp.float32)
        return carry
    lax.fori_loop(0, K, zrow, 0)
    for t in range(RPT // K):
        pltpu.sync_copy(msg0_v, acc.at[pl.ds(sid * RPT + t * K, K)])
    ztail = RPT - (RPT // K) * K
    pltpu.sync_copy(msg0_v.at[pl.ds(0, ztail)],
                    acc.at[pl.ds(sid * RPT + (RPT // K) * K, ztail)])

    r0 = sid * RPT
    pltpu.sync_copy(x_hbm.at[pl.ds(r0, RPT), pl.ds(cid * DH, DH)],
                    xtab.at[pl.ds(r0, RPT)])
    plsc.subcore_barrier()

    fbase = sid * EPT2

    def start(jb, jl, rows_v, filt_v, sem_g, sem_f):
        o = jl * KH
        pltpu.async_copy(xtab.at[srcl_v.at[pl.ds(o, KH)]],
                         rows_v.at[pl.ds(0, KH)], sem_g)
        pltpu.async_copy(xtab.at[srch_v.at[pl.ds(o, KH)]],
                         rows_v.at[pl.ds(KH, KH)], sem_g)
        pltpu.async_copy(
            filt_hbm.at[cid, pl.ds(fbase + (jb + jl) * KH, KH)],
            filt_v, sem_f)

    def wait_gf(rows_v, filt_v, sem_g, sem_f):
        pltpu.make_async_copy(xtab.at[srcl_v.at[pl.ds(0, KH)]],
                              rows_v.at[pl.ds(0, KH)], sem_g).wait()
        pltpu.make_async_copy(xtab.at[srcl_v.at[pl.ds(0, KH)]],
                              rows_v.at[pl.ds(KH, KH)], sem_g).wait()
        pltpu.make_async_copy(filt_hbm.at[cid, pl.ds(fbase, KH)],
                              filt_v, sem_f).wait()

    def wait_s(msg_v, sem_s):
        pltpu.make_async_copy(msg_v.at[pl.ds(0, KH)],
                              acc.at[dstl_v.at[pl.ds(0, KH)]], sem_s).wait()
        pltpu.make_async_copy(msg_v.at[pl.ds(KH, KH)],
                              acc.at[dstl_v.at[pl.ds(0, KH)]], sem_s).wait()

    def finish(jb, jl, first, rows_v, filt_v, msg_v, sem_g, sem_f, sem_s):
        wait_gf(rows_v, filt_v, sem_g, sem_f)

        @pl.when(jnp.logical_not(first))
        def _():
            wait_s(msg_v, sem_s)

        @plsc.parallel_loop(0, KH, unroll=4)
        def mrow(r):
            for c in range(DH // 16):
                s = pl.ds(c * 16, 16)
                msg_v[r, s] = filt_v[r, s] * rows_v[r, s]
            for c in range(DH // 16):
                s = pl.ds(c * 16, 16)
                msg_v[KH + r, s] = (
                    filt_v[r, pl.ds(DH + c * 16, 16)] * rows_v[KH + r, s])

        o = jl * KH
        pltpu.async_copy(msg_v.at[pl.ds(0, KH)],
                         acc.at[dstl_v.at[pl.ds(o, KH)]], sem_s, add=True)
        pltpu.async_copy(msg_v.at[pl.ds(KH, KH)],
                         acc.at[dsth_v.at[pl.ds(o, KH)]], sem_s, add=True)

    for p in range(NPASS):
        jb = p * TPP
        ib = fbase + jb * KH
        pltpu.sync_copy(ei_hbm.at[0, pl.ds(ib, IPP)], srcl_v)
        pltpu.sync_copy(ei_hbm.at[0, pl.ds(E2 + ib, IPP)], srch_v)
        pltpu.sync_copy(ei_hbm.at[1, pl.ds(ib, IPP)], dstl_v)
        pltpu.sync_copy(ei_hbm.at[1, pl.ds(E2 + ib, IPP)], dsth_v)
        start(jb, 0, rows0_v, filt0_v, sem_g0, sem_f0)
        start(jb, 1, rows1_v, filt1_v, sem_g1, sem_f1)

        def pipe(j2, carry):
            a = 2 * j2
            finish(jb, a, j2 == 0, rows0_v, filt0_v, msg0_v,
                   sem_g0, sem_f0, sem_s0)
            start(jb, jnp.minimum(a + 2, TPP - 1), rows0_v, filt0_v,
                  sem_g0, sem_f0)
            finish(jb, a + 1, j2 == 0, rows1_v, filt1_v, msg1_v,
                   sem_g1, sem_f1, sem_s1)
            start(jb, jnp.minimum(a + 3, TPP - 1), rows1_v, filt1_v,
                  sem_g1, sem_f1)
            return carry
        lax.fori_loop(0, TPP // 2, pipe, 0)
        wait_gf(rows0_v, filt0_v, sem_g0, sem_f0)
        wait_gf(rows1_v, filt1_v, sem_g1, sem_f1)
        wait_s(msg0_v, sem_s0)
        wait_s(msg1_v, sem_s1)
    plsc.subcore_barrier()

    pltpu.sync_copy(acc.at[pl.ds(sid * RPT, RPT)],
                    out_hbm.at[pl.ds(sid * RPT, RPT), pl.ds(cid * DH, DH)])


_sc_scatter = functools.partial(
    pl.kernel,
    out_type=jax.ShapeDtypeStruct((N, D), jnp.float32),
    mesh=plsc.VectorSubcoreMesh(core_axis_name="c", subcore_axis_name="s"),
    compiler_params=pltpu.CompilerParams(use_tc_tiling_on_sc=False),
    scratch_types=[
        pltpu.VMEM((IPP,), jnp.int32),
        pltpu.VMEM((IPP,), jnp.int32),
        pltpu.VMEM((IPP,), jnp.int32),
        pltpu.VMEM((IPP,), jnp.int32),
        pltpu.VMEM((K, DH), jnp.float32),
        pltpu.VMEM((K, DH), jnp.float32),
        pltpu.VMEM((KH, D), jnp.float32),
        pltpu.VMEM((KH, D), jnp.float32),
        pltpu.VMEM((K, DH), jnp.float32),
        pltpu.VMEM((K, DH), jnp.float32),
        pltpu.VMEM_SHARED((N, DH), jnp.float32),
        pltpu.VMEM_SHARED((N, DH), jnp.float32),
        pltpu.SemaphoreType.DMA,
        pltpu.SemaphoreType.DMA,
        pltpu.SemaphoreType.DMA,
        pltpu.SemaphoreType.DMA,
        pltpu.SemaphoreType.DMA,
        pltpu.SemaphoreType.DMA,
    ],
)(_sc_body)


def kernel(x, edge_index, edge_basis, W, b):
    filt2 = _compute_filt(edge_basis, W, b)
    return _sc_scatter(x, edge_index, filt2)
